# Initial kernel scaffold; baseline (speedup 1.0000x reference)
import functools

import jax
import jax.numpy as jnp
from jax import lax
from jax.experimental import pallas as pl
from jax.experimental.pallas import tpu as pltpu
from jax.experimental.pallas import tpu_sc as plsc

N_NUM = 13
N_CAT = 26
CARD = 1000
D = 128
B = 4096
N_TOK = N_NUM + N_CAT

_INFO = plsc.get_sparse_core_info()
_NC = _INFO.num_cores
_NS = _INFO.num_subcores
_NW = _NC * _NS
_ROWS_PER_W = B // _NW
_CHUNK = 8
_N_CHUNKS = _ROWS_PER_W // _CHUNK
_GAT = _CHUNK * N_CAT
_NVEC = D // 16


def _sc_body(xnum_hbm, xcat_hbm, w_hbm, nb_hbm, tab_hbm, cb_hbm, out_hbm,
             w_v, nbias_v, cb_v, xnum_v, xcat_v, idx_v, gat_v, out_v, sem):
    wid = lax.axis_index("s") * _NC + lax.axis_index("c")
    base = wid * _ROWS_PER_W

    pltpu.sync_copy(w_hbm, w_v)
    pltpu.sync_copy(nb_hbm, nbias_v)
    pltpu.sync_copy(cb_hbm, cb_v)

    lanes = lax.iota(jnp.int32, 16)

    def chunk_body(c, carry):
        row0 = base + c * _CHUNK
        pltpu.sync_copy(xnum_hbm.at[pl.ds(row0 * N_NUM, _CHUNK * N_NUM)],
                        xnum_v)
        pltpu.sync_copy(xcat_hbm.at[pl.ds(row0 * N_CAT, _GAT)], xcat_v)

        def idx_body(i, carry2):
            s = pl.ds(i * 16, 16)
            p = lanes + i * 16
            f = lax.rem(p, N_CAT)
            idx_v[s] = xcat_v[s] + f * CARD
            return carry2

        lax.fori_loop(0, _GAT // 16, idx_body, 0)

        pltpu.async_copy(tab_hbm.at[idx_v], gat_v, sem).wait()

        def r_body(r, carry2):
            def f_num(f, carry3):
                xs = xnum_v[r * N_NUM + f]
                for v in range(_NVEC):
                    s = pl.ds(v * 16, 16)
                    out_v[r, f, s] = xs * w_v[f, s] + nbias_v[f, s]
                return carry3

            lax.fori_loop(0, N_NUM, f_num, 0)

            def f_cat(f, carry3):
                g = r * N_CAT + f
                for v in range(_NVEC):
                    s = pl.ds(v * 16, 16)
                    out_v[r, N_NUM + f, s] = gat_v[g, s] + cb_v[f, s]
                return carry3

            lax.fori_loop(0, N_CAT, f_cat, 0)
            return carry2

        lax.fori_loop(0, _CHUNK, r_body, 0)

        pltpu.sync_copy(out_v, out_hbm.at[pl.ds(row0, _CHUNK)])
        return carry

    lax.fori_loop(0, _N_CHUNKS, chunk_body, 0)


@jax.jit
def _sc_call(x_num_flat, x_cat_flat, num_weight, num_bias, cat_table,
             cat_bias):
    mesh = plsc.VectorSubcoreMesh(core_axis_name="c", subcore_axis_name="s")
    run = pl.kernel(
        _sc_body,
        mesh=mesh,
        out_type=jax.ShapeDtypeStruct((B, N_TOK, D), jnp.float32),
        scratch_types=[
            pltpu.VMEM((N_NUM, D), jnp.float32),
            pltpu.VMEM((N_NUM, D), jnp.float32),
            pltpu.VMEM((N_CAT, D), jnp.float32),
            pltpu.VMEM((_CHUNK * N_NUM,), jnp.float32),
            pltpu.VMEM((_GAT,), jnp.int32),
            pltpu.VMEM((_GAT,), jnp.int32),
            pltpu.VMEM((_GAT, D), jnp.float32),
            pltpu.VMEM((_CHUNK, N_TOK, D), jnp.float32),
            pltpu.SemaphoreType.DMA,
        ],
    )
    return run(x_num_flat, x_cat_flat, num_weight, num_bias, cat_table,
               cat_bias)


def kernel(x_num, x_cat, num_weight, num_bias, cat_table, cat_bias):
    x_num_flat = x_num.reshape(-1)
    x_cat_flat = x_cat.astype(jnp.int32).reshape(-1)
    return _sc_call(x_num_flat, x_cat_flat, num_weight, num_bias, cat_table,
                    cat_bias)


# speedup vs baseline: 1.6582x; 1.6582x over previous
import functools

import jax
import jax.numpy as jnp
from jax import lax
from jax.experimental import pallas as pl
from jax.experimental.pallas import tpu as pltpu
from jax.experimental.pallas import tpu_sc as plsc

N_NUM = 13
N_CAT = 26
CARD = 1000
D = 128
B = 4096
N_TOK = N_NUM + N_CAT

_INFO = plsc.get_sparse_core_info()
_NC = _INFO.num_cores
_NS = _INFO.num_subcores
_NW = _NC * _NS
_ROWS_PER_W = B // _NW
_CHUNK = 8
_N_CHUNKS = _ROWS_PER_W // _CHUNK
_GAT = _CHUNK * N_CAT
_NVEC = D // 16


def _sc_body(xnum_hbm, xcat_hbm, w_hbm, nb_hbm, tab_hbm, cb_hbm, out_hbm,
             w_v, nbias_v, cb_v, xnum_v, xcat_v, idx_v, gat_v, out_v, sem):
    wid = lax.axis_index("s") * _NC + lax.axis_index("c")
    base = wid * _ROWS_PER_W

    pltpu.sync_copy(w_hbm, w_v)
    pltpu.sync_copy(nb_hbm, nbias_v)
    pltpu.sync_copy(cb_hbm, cb_v)

    lanes = lax.iota(jnp.int32, 16)

    def chunk_body(c, carry):
        row0 = base + c * _CHUNK
        pltpu.sync_copy(xnum_hbm.at[pl.ds(row0, _CHUNK)], xnum_v)
        pltpu.sync_copy(xcat_hbm.at[pl.ds(row0 * N_CAT, _GAT)], xcat_v)

        def idx_body(i, carry2):
            s = pl.ds(i * 16, 16)
            p = lanes + i * 16
            f = lax.rem(p, N_CAT)
            idx_v[s] = xcat_v[s] + f * CARD
            return carry2

        lax.fori_loop(0, _GAT // 16, idx_body, 0)

        pltpu.async_copy(tab_hbm.at[idx_v], gat_v, sem).wait()

        def r_body(r, carry2):
            xrow = xnum_v[r]
            for f in range(N_NUM):
                xs = xrow[f]
                for v in range(_NVEC):
                    s = pl.ds(v * 16, 16)
                    out_v[r, f, s] = xs * w_v[f, s] + nbias_v[f, s]

            def f_cat(f, carry3):
                g = r * N_CAT + f
                for v in range(_NVEC):
                    s = pl.ds(v * 16, 16)
                    out_v[r, N_NUM + f, s] = gat_v[g, s] + cb_v[f, s]
                return carry3

            lax.fori_loop(0, N_CAT, f_cat, 0)
            return carry2

        lax.fori_loop(0, _CHUNK, r_body, 0)

        pltpu.sync_copy(out_v, out_hbm.at[pl.ds(row0, _CHUNK)])
        return carry

    lax.fori_loop(0, _N_CHUNKS, chunk_body, 0)


@jax.jit
def _sc_call(x_num_flat, x_cat_flat, num_weight, num_bias, cat_table,
             cat_bias):
    mesh = plsc.VectorSubcoreMesh(core_axis_name="c", subcore_axis_name="s")
    run = pl.kernel(
        _sc_body,
        mesh=mesh,
        out_type=jax.ShapeDtypeStruct((B, N_TOK, D), jnp.float32),
        scratch_types=[
            pltpu.VMEM((N_NUM, D), jnp.float32),
            pltpu.VMEM((N_NUM, D), jnp.float32),
            pltpu.VMEM((N_CAT, D), jnp.float32),
            pltpu.VMEM((_CHUNK, 16), jnp.float32),
            pltpu.VMEM((_GAT,), jnp.int32),
            pltpu.VMEM((_GAT,), jnp.int32),
            pltpu.VMEM((_GAT, D), jnp.float32),
            pltpu.VMEM((_CHUNK, N_TOK, D), jnp.float32),
            pltpu.SemaphoreType.DMA,
        ],
    )
    return run(x_num_flat, x_cat_flat, num_weight, num_bias, cat_table,
               cat_bias)


def kernel(x_num, x_cat, num_weight, num_bias, cat_table, cat_bias):
    x_num_pad = jnp.pad(x_num, ((0, 0), (0, 16 - N_NUM)))
    x_cat_flat = x_cat.astype(jnp.int32).reshape(-1)
    return _sc_call(x_num_pad, x_cat_flat, num_weight, num_bias, cat_table,
                    cat_bias)


# speedup vs baseline: 1.8923x; 1.1412x over previous
import jax
import jax.numpy as jnp
from jax import lax
from jax.experimental import pallas as pl
from jax.experimental.pallas import tpu as pltpu
from jax.experimental.pallas import tpu_sc as plsc

N_NUM = 13
N_CAT = 26
CARD = 1000
D = 128
B = 4096
N_TOK = N_NUM + N_CAT

_INFO = plsc.get_sparse_core_info()
_NC = _INFO.num_cores
_NS = _INFO.num_subcores
_NW = _NC * _NS
_RPW = B // _NW
_CAT_PER_W = _RPW * N_CAT
_G = 128
_NGRP = _CAT_PER_W // _G


def _fold_body(tab_ref, bias_ref, out_ref):
    out_ref[...] = tab_ref[...] + bias_ref[0]


def _fold_table(cat_table, cat_bias):
    return pl.pallas_call(
        _fold_body,
        grid=(N_CAT,),
        in_specs=[
            pl.BlockSpec((CARD, D), lambda i: (i, 0)),
            pl.BlockSpec((1, 1, D), lambda i: (i, 0, 0)),
        ],
        out_specs=pl.BlockSpec((CARD, D), lambda i: (i, 0)),
        out_shape=jax.ShapeDtypeStruct((N_CAT * CARD, D), jnp.float32),
    )(cat_table, cat_bias[:, None, :])


def _sc_body(xnum_hbm, xcat_hbm, w_hbm, nb_hbm, tab_hbm, out_hbm,
             w_v, nb_v, xnum_v, xcat_v, src_all, dst_all, dstn_all,
             gat_v, num0_v, num1_v, sem_g, sem_s, sem_n0, sem_n1):
    wid = lax.axis_index("s") * _NC + lax.axis_index("c")
    base = wid * _RPW

    pltpu.sync_copy(w_hbm, w_v)
    pltpu.sync_copy(nb_hbm, nb_v)
    pltpu.sync_copy(xnum_hbm.at[pl.ds(base, _RPW)], xnum_v)
    pltpu.sync_copy(xcat_hbm.at[pl.ds(base * N_CAT, _CAT_PER_W)], xcat_v)

    lanes = lax.iota(jnp.int32, 16)

    def fill_cat(j, carry):
        for i in range(8):
            p = j * _G + i * 16 + lanes
            r = lax.div(p, jnp.int32(N_CAT))
            f = p - r * N_CAT
            code = xcat_v[pl.ds(j * _G + i * 16, 16)]
            src_all[j, pl.ds(i * 16, 16)] = code + f * CARD
            dst_all[j, pl.ds(i * 16, 16)] = (base + r) * N_TOK + N_NUM + f
        return carry

    lax.fori_loop(0, _NGRP, fill_cat, 0)

    def fill_num(f, carry):
        for i in range(8):
            rvec = i * 16 + lanes
            dstn_all[f, pl.ds(i * 16, 16)] = (base + rvec) * N_TOK + f
        return carry

    lax.fori_loop(0, N_NUM, fill_num, 0)

    handles = [None, None]
    for f in range(N_NUM):
        buf = num0_v if f % 2 == 0 else num1_v
        sem = sem_n0 if f % 2 == 0 else sem_n1
        if handles[f % 2] is not None:
            handles[f % 2].wait()
        def body_r(r, carry, f=f, buf=buf):
            xs = xnum_v[r][f]
            for i in range(8):
                s = pl.ds(i * 16, 16)
                buf[r, s] = xs * w_v[f, s] + nb_v[f, s]
            return carry

        lax.fori_loop(0, _RPW, body_r, 0)
        handles[f % 2] = pltpu.async_copy(
            buf, out_hbm.at[dstn_all.at[f]], sem)

    def cat_body(j, carry):
        pltpu.async_copy(tab_hbm.at[src_all.at[j]], gat_v, sem_g).wait()
        pltpu.async_copy(gat_v, out_hbm.at[dst_all.at[j]], sem_s).wait()
        return carry

    lax.fori_loop(0, _NGRP, cat_body, 0)

    for h in handles:
        if h is not None:
            h.wait()


@jax.jit
def _run(x_num_pad, x_cat_flat, num_weight, num_bias, cat_table, cat_bias):
    folded = _fold_table(cat_table, cat_bias)
    mesh = plsc.VectorSubcoreMesh(core_axis_name="c", subcore_axis_name="s")
    sc = pl.kernel(
        _sc_body,
        mesh=mesh,
        out_type=jax.ShapeDtypeStruct((B * N_TOK, D), jnp.float32),
        scratch_types=[
            pltpu.VMEM((N_NUM, D), jnp.float32),
            pltpu.VMEM((N_NUM, D), jnp.float32),
            pltpu.VMEM((_RPW, 16), jnp.float32),
            pltpu.VMEM((_CAT_PER_W,), jnp.int32),
            pltpu.VMEM((_NGRP, _G), jnp.int32),
            pltpu.VMEM((_NGRP, _G), jnp.int32),
            pltpu.VMEM((N_NUM, _RPW), jnp.int32),
            pltpu.VMEM((_G, D), jnp.float32),
            pltpu.VMEM((_RPW, D), jnp.float32),
            pltpu.VMEM((_RPW, D), jnp.float32),
            pltpu.SemaphoreType.DMA,
            pltpu.SemaphoreType.DMA,
            pltpu.SemaphoreType.DMA,
            pltpu.SemaphoreType.DMA,
        ],
    )
    out_flat = sc(x_num_pad, x_cat_flat, num_weight, num_bias, folded)
    return out_flat.reshape(B, N_TOK, D)


def kernel(x_num, x_cat, num_weight, num_bias, cat_table, cat_bias):
    x_num_pad = jnp.pad(x_num, ((0, 0), (0, 16 - N_NUM)))
    x_cat_flat = x_cat.astype(jnp.int32).reshape(-1)
    return _run(x_num_pad, x_cat_flat, num_weight, num_bias, cat_table,
                cat_bias)


# speedup vs baseline: 3.3627x; 1.7770x over previous
import jax
import jax.numpy as jnp
from jax import lax
from jax.experimental import pallas as pl
from jax.experimental.pallas import tpu as pltpu
from jax.experimental.pallas import tpu_sc as plsc

N_NUM = 13
N_CAT = 26
CARD = 1000
D = 128
B = 4096
N_TOK = N_NUM + N_CAT

_INFO = plsc.get_sparse_core_info()
_NC = _INFO.num_cores
_NS = _INFO.num_subcores
_NW = _NC * _NS
_RPW = B // _NW
_CHUNK = 8
_NSUPER = _RPW // (2 * _CHUNK)
_CODES = _CHUNK * N_CAT


def _fold_body(tab_ref, bias_ref, out_ref):
    out_ref[...] = tab_ref[...] + bias_ref[0]


def _fold_table(cat_table, cat_bias):
    return pl.pallas_call(
        _fold_body,
        grid=(N_CAT,),
        in_specs=[
            pl.BlockSpec((CARD, D), lambda i: (i, 0)),
            pl.BlockSpec((1, 1, D), lambda i: (i, 0, 0)),
        ],
        out_specs=pl.BlockSpec((CARD, D), lambda i: (i, 0)),
        out_shape=jax.ShapeDtypeStruct((N_CAT * CARD, D), jnp.float32),
    )(cat_table, cat_bias[:, None, :])


def _sc_body(xnum_hbm, xcat_hbm, w_hbm, nb_hbm, tab_hbm, out_hbm,
             w_v, nb_v, xnum_v, xcat_v, idx0_v, idx1_v, out0_v, out1_v,
             sem_g0, sem_g1, sem_o0, sem_o1):
    wid = lax.axis_index("s") * _NC + lax.axis_index("c")
    base = wid * _RPW

    pltpu.sync_copy(w_hbm, w_v)
    pltpu.sync_copy(nb_hbm, nb_v)
    pltpu.sync_copy(xnum_hbm.at[pl.ds(base, _RPW)], xnum_v)
    pltpu.sync_copy(xcat_hbm.at[pl.ds(base, _RPW)], xcat_v)

    lanes = lax.iota(jnp.int32, 16)
    off_lo = lanes * CARD
    off_hi = jnp.minimum(lanes + 16, N_CAT - 1) * CARD

    def do_chunk(c, idx_v, out_v, sem_g, sem_o, first):
        row0 = base + c * _CHUNK

        @pl.when(jnp.logical_not(first))
        def _():
            pltpu.make_async_copy(
                out_v, out_hbm.at[pl.ds(row0, _CHUNK)], sem_o).wait()

        for rr in range(_CHUNK):
            s_lo = pl.ds(0, 16)
            s_hi = pl.ds(16, 16)
            idx_v[rr, s_lo] = xcat_v[c * _CHUNK + rr, s_lo] + off_lo
            idx_v[rr, s_hi] = xcat_v[c * _CHUNK + rr, s_hi] + off_hi

        handles = []
        for rr in range(_CHUNK):
            handles.append(pltpu.async_copy(
                tab_hbm.at[idx_v.at[rr, pl.ds(0, N_CAT)]],
                out_v.at[rr, pl.ds(N_NUM, N_CAT)],
                sem_g))

        def num_fill(rr, carry):
            xrow = xnum_v[c * _CHUNK + rr]
            for f in range(N_NUM):
                xs = xrow[f]
                for i in range(D // 16):
                    s = pl.ds(i * 16, 16)
                    out_v[rr, f, s] = xs * w_v[f, s] + nb_v[f, s]
            return carry

        lax.fori_loop(0, _CHUNK, num_fill, 0)

        for h in handles:
            h.wait()

        pltpu.async_copy(out_v, out_hbm.at[pl.ds(row0, _CHUNK)], sem_o)

    def super_step(cs, carry):
        do_chunk(cs * 2, idx0_v, out0_v, sem_g0, sem_o0, cs == 0)
        do_chunk(cs * 2 + 1, idx1_v, out1_v, sem_g1, sem_o1, cs == 0)
        return carry

    lax.fori_loop(0, _NSUPER, super_step, 0)

    pltpu.make_async_copy(
        out0_v, out_hbm.at[pl.ds(base, _CHUNK)], sem_o0).wait()
    pltpu.make_async_copy(
        out1_v, out_hbm.at[pl.ds(base, _CHUNK)], sem_o1).wait()


@jax.jit
def _run(x_num_pad, x_cat_flat, num_weight, num_bias, cat_table, cat_bias):
    folded = _fold_table(cat_table, cat_bias)
    mesh = plsc.VectorSubcoreMesh(core_axis_name="c", subcore_axis_name="s")
    sc = pl.kernel(
        _sc_body,
        mesh=mesh,
        out_type=jax.ShapeDtypeStruct((B, N_TOK, D), jnp.float32),
        scratch_types=[
            pltpu.VMEM((N_NUM, D), jnp.float32),
            pltpu.VMEM((N_NUM, D), jnp.float32),
            pltpu.VMEM((_RPW, 16), jnp.float32),
            pltpu.VMEM((_RPW, 32), jnp.int32),
            pltpu.VMEM((_CHUNK, 32), jnp.int32),
            pltpu.VMEM((_CHUNK, 32), jnp.int32),
            pltpu.VMEM((_CHUNK, N_TOK, D), jnp.float32),
            pltpu.VMEM((_CHUNK, N_TOK, D), jnp.float32),
            pltpu.SemaphoreType.DMA,
            pltpu.SemaphoreType.DMA,
            pltpu.SemaphoreType.DMA,
            pltpu.SemaphoreType.DMA,
        ],
    )
    return sc(x_num_pad, x_cat_flat, num_weight, num_bias, folded)


def kernel(x_num, x_cat, num_weight, num_bias, cat_table, cat_bias):
    x_num_pad = jnp.pad(x_num, ((0, 0), (0, 16 - N_NUM)))
    x_cat_pad = jnp.pad(x_cat.astype(jnp.int32),
                        ((0, 0), (0, 32 - N_CAT)))
    return _run(x_num_pad, x_cat_pad, num_weight, num_bias, cat_table,
                cat_bias)
